# baseline (speedup 1.0000x reference)
import functools

import jax
import jax.numpy as jnp
from jax import lax
from jax.experimental import pallas as pl
from jax.experimental.pallas import tpu as pltpu
from jax.experimental.pallas import tpu_sc as plsc

NCL = 32
B = 16384

_info = plsc.get_sparse_core_info()
_NC, _NS, _L = _info.num_cores, _info.num_subcores, _info.num_lanes
_NW = _NC * _NS
_BPW = B // _NW


def _tec_body(ixs_hbm, w_hbm, out_hbm, idx_v, rows_v, rows_t, sem):
    wid = lax.axis_index("s") * _NC + lax.axis_index("c")
    base = wid * _BPW
    pltpu.sync_copy(ixs_hbm.at[pl.ds(base, _BPW)], idx_v)
    pltpu.async_copy(w_hbm.at[idx_v], rows_v, sem).wait()
    iota = lax.iota(jnp.int32, _L)

    def body(j, carry):
        lo = rows_v[j, pl.ds(0, _L)]
        hi = rows_v[j, pl.ds(_L, _L)]
        jv = jnp.full((_L,), j, jnp.int32)
        plsc.store_scatter(rows_t, [iota, jv], lo)
        plsc.store_scatter(rows_t, [iota + _L, jv], hi)
        return carry

    lax.fori_loop(0, _BPW, body, 0)
    pltpu.sync_copy(rows_t, out_hbm.at[:, pl.ds(base, _BPW)])


def kernel(variantxgene_ixs, weight):
    f = functools.partial(
        pl.kernel,
        mesh=plsc.VectorSubcoreMesh(core_axis_name="c", subcore_axis_name="s"),
        compiler_params=pltpu.CompilerParams(
            needs_layout_passes=False, use_tc_tiling_on_sc=False),
        out_type=jax.ShapeDtypeStruct((NCL, B), jnp.float32),
        scratch_types=[
            pltpu.VMEM((_BPW,), jnp.int32),
            pltpu.VMEM((_BPW, NCL), jnp.float32),
            pltpu.VMEM((NCL, _BPW), jnp.float32),
            pltpu.SemaphoreType.DMA,
        ],
    )(_tec_body)
    return f(variantxgene_ixs, weight)
